# baseline (speedup 1.0000x reference)
import jax
import jax.numpy as jnp
from jax import lax
from jax.experimental import pallas as pl
from jax.experimental.pallas import tpu as pltpu
from jax.experimental.pallas import tpu_sc as plsc

F32 = jnp.float32
NC = 2
NS = 16
NW = NC * NS
BE = 96
ROWW = 40
BLK = 1000


def _lr(x, s):
    return jnp.where(x > 0, x, s * x)


def _tc1_body(des_r, tw_r, np_r, cp_r, Wd, bd, Wt, bt, Wn, bn, Wc, bc,
              Win, bin_, Wg1, a1s, a1d,
              h1_o, ht0_o, ht1_o, ht2_o, ht3_o, as_o, ad_o, amax_o):
    d = _lr(jnp.dot(des_r[...], Wd[...], preferred_element_type=F32) + bd[...], 0.01)
    t = _lr(jnp.dot(tw_r[...], Wt[...], preferred_element_type=F32) + bt[...], 0.01)
    npp = _lr(jnp.dot(np_r[...], Wn[...], preferred_element_type=F32) + bn[...], 0.01)
    c = _lr(jnp.dot(cp_r[...], Wc[...], preferred_element_type=F32) + bc[...], 0.01)
    x = jnp.concatenate([d, t, npp, c], axis=1)
    x = _lr(jnp.dot(x, Win[...], preferred_element_type=F32) + bin_[...], 0.01)
    h1 = jnp.dot(x, Wg1[...], preferred_element_type=F32)
    h1_o[...] = h1
    h1r = h1.reshape(-1, 4, 32)
    a_s = jnp.sum(h1r * a1s[...][None], axis=-1)
    a_d = jnp.sum(h1r * a1d[...][None], axis=-1)
    as_o[...] = a_s
    ad_o[...] = a_d
    one = jnp.ones((h1.shape[0], 1), F32)
    zer = jnp.zeros((h1.shape[0], 6), F32)
    for h, ref in enumerate((ht0_o, ht1_o, ht2_o, ht3_o)):
        ref[...] = jnp.concatenate(
            [one, a_s[:, h:h + 1], zer, h1[:, 32 * h:32 * h + 32]], axis=1)

    @pl.when(pl.program_id(0) == 0)
    def _():
        amax_o[...] = jnp.full((1, 4), -jnp.inf, F32)

    amax_o[...] = jnp.maximum(amax_o[...], jnp.max(a_s, axis=0)[None])


def _tc1(des, tweet, num_prop, cat_prop, Wd, bd, Wt, bt, Wn, bn, Wc, bc,
         Win, bin_, Wg1, a1s, a1d):
    n = des.shape[0]
    grid = n // BLK
    row = lambda i: (i, 0)
    full = lambda i: (0, 0)
    return pl.pallas_call(
        _tc1_body,
        grid=(grid,),
        in_specs=[
            pl.BlockSpec((BLK, 768), row), pl.BlockSpec((BLK, 768), row),
            pl.BlockSpec((BLK, 5), row), pl.BlockSpec((BLK, 3), row),
            pl.BlockSpec((768, 32), full), pl.BlockSpec((1, 32), full),
            pl.BlockSpec((768, 32), full), pl.BlockSpec((1, 32), full),
            pl.BlockSpec((5, 32), full), pl.BlockSpec((1, 32), full),
            pl.BlockSpec((3, 32), full), pl.BlockSpec((1, 32), full),
            pl.BlockSpec((128, 128), full), pl.BlockSpec((1, 128), full),
            pl.BlockSpec((128, 128), full),
            pl.BlockSpec((4, 32), full), pl.BlockSpec((4, 32), full),
        ],
        out_specs=[
            pl.BlockSpec((BLK, 128), row),
            pl.BlockSpec((BLK, ROWW), row), pl.BlockSpec((BLK, ROWW), row),
            pl.BlockSpec((BLK, ROWW), row), pl.BlockSpec((BLK, ROWW), row),
            pl.BlockSpec((BLK, 4), row), pl.BlockSpec((BLK, 4), row),
            pl.BlockSpec((1, 4), full),
        ],
        out_shape=[
            jax.ShapeDtypeStruct((n, 128), F32),
            jax.ShapeDtypeStruct((n, ROWW), F32), jax.ShapeDtypeStruct((n, ROWW), F32),
            jax.ShapeDtypeStruct((n, ROWW), F32), jax.ShapeDtypeStruct((n, ROWW), F32),
            jax.ShapeDtypeStruct((n, 4), F32), jax.ShapeDtypeStruct((n, 4), F32),
            jax.ShapeDtypeStruct((1, 4), F32),
        ],
    )(des, tweet, num_prop, cat_prop, Wd, bd, Wt, bt, Wn, bn, Wc, bc,
      Win, bin_, Wg1, a1s, a1d)


def _make_sc(n, npad, rpt, ept, nblk):
    def body(src_hbm, dst_hbm, ad0, ad1, ad2, ad3,
             ht0, ht1, ht2, ht3, a_hbm, z_hbm,
             out_hbm, a_v, src_v, dst_v, adb, rows_v, u_sh,
             sem1, sem3):
        cid = lax.axis_index("c")
        sid = lax.axis_index("s")
        wid = sid * NC + cid
        hts = (ht0, ht1, ht2, ht3)
        ads = (ad0, ad1, ad2, ad3)
        lt8 = jnp.arange(16, dtype=jnp.int32) < 8
        one16 = jnp.ones((16,), F32)
        for p in range(4):
            pltpu.sync_copy(a_hbm.at[p], a_v)
            pltpu.sync_copy(z_hbm, u_sh.at[pl.ds(sid * rpt, rpt)])
            plsc.subcore_barrier()
            av = a_v[...]

            def blk(b, carry):
                base = wid * ept + b * BE
                pltpu.sync_copy(src_hbm.at[pl.ds(base, BE)], src_v)
                pltpu.sync_copy(dst_hbm.at[pl.ds(base, BE)], dst_v)
                c1 = pltpu.async_copy(hts[p].at[src_v], rows_v, sem1)
                c3 = pltpu.async_copy(ads[p].at[dst_v], adb, sem3)
                c1.wait()
                c3.wait()
                def edge(e, ecarry):
                    r0 = rows_v[e, pl.ds(0, 16)]
                    as16 = jnp.full((16,), r0[1], F32)
                    ad16 = adb[e, pl.ds(0, 16)]
                    z = as16 + ad16
                    e16 = jnp.where(z > 0, z, 0.2 * z)
                    zm = av + ad16
                    m16 = jnp.where(zm > 0, zm, 0.2 * zm)
                    wv = jnp.exp(e16 - m16)
                    wt = jnp.where(lt8, one16, wv)
                    rows_v[e, pl.ds(0, 16)] = r0 * wv
                    rows_v[e, pl.ds(16, 16)] = rows_v[e, pl.ds(16, 16)] * wv
                    rows_v[e, pl.ds(24, 16)] = rows_v[e, pl.ds(24, 16)] * wt
                    return ecarry

                lax.fori_loop(0, BE, edge, 0, unroll=4)
                pltpu.sync_copy(rows_v, u_sh.at[dst_v], add=True)
                return carry

            lax.fori_loop(0, nblk, blk, 0)
            plsc.subcore_barrier()
            pltpu.sync_copy(u_sh.at[pl.ds(sid * rpt, rpt)],
                            out_hbm.at[p, cid, pl.ds(sid * rpt, rpt)])
            plsc.subcore_barrier()

    return pl.kernel(
        body,
        out_type=jax.ShapeDtypeStruct((4, NC, npad, ROWW), F32),
        mesh=plsc.VectorSubcoreMesh(core_axis_name="c", subcore_axis_name="s",
                                    num_cores=NC, num_subcores=NS),
        compiler_params=pltpu.CompilerParams(needs_layout_passes=False,
                                             use_tc_tiling_on_sc=False),
        scratch_types=[
            pltpu.VMEM((16,), F32),
            pltpu.VMEM((BE,), jnp.int32), pltpu.VMEM((BE,), jnp.int32),
            pltpu.VMEM((BE, 16), F32),
            pltpu.VMEM((BE, ROWW), F32),
            pltpu.VMEM_SHARED((npad, ROWW), F32),
            pltpu.SemaphoreType.DMA, pltpu.SemaphoreType.DMA,
        ],
    )


def _tc2_body(u_r, h1_r, as_r, ad_r, A_r, bg1_r, Wg2, a2s_r, a2d_r,
              h2_o, ht0_o, ht1_o, ht2_o, ht3_o, a2so, a2do, amax_o):
    asn = as_r[...]
    adn = ad_r[...]
    z = asn + adn
    e = jnp.where(z > 0, z, 0.2 * z)
    zm = A_r[...] + adn
    m = jnp.where(zm > 0, zm, 0.2 * zm)
    ws = jnp.exp(e - m)
    u = u_r[...]
    h1 = h1_r[...]
    cols = []
    for h in range(4):
        uh = u[2 * h] + u[2 * h + 1]
        wsh = ws[:, h:h + 1]
        num = uh[:, 8:40] + wsh * h1[:, 32 * h:32 * h + 32]
        den = uh[:, 0:1] + wsh + 1e-16
        cols.append(num / den)
    x1 = jnp.concatenate(cols, axis=1) + bg1_r[...]
    h2 = jnp.dot(x1, Wg2[...], preferred_element_type=F32)
    h2_o[...] = h2
    a2s = jnp.sum(h2 * a2s_r[...], axis=1, keepdims=True)
    a2d = jnp.sum(h2 * a2d_r[...], axis=1, keepdims=True)
    a2so[...] = a2s
    a2do[...] = a2d
    one = jnp.ones((h2.shape[0], 1), F32)
    zer = jnp.zeros((h2.shape[0], 6), F32)
    for c, ref in enumerate((ht0_o, ht1_o, ht2_o, ht3_o)):
        ref[...] = jnp.concatenate(
            [one, a2s, zer, h2[:, 32 * c:32 * c + 32]], axis=1)

    @pl.when(pl.program_id(0) == 0)
    def _():
        amax_o[...] = jnp.full((1, 1), -jnp.inf, F32)

    amax_o[...] = jnp.maximum(amax_o[...], jnp.max(a2s, axis=0, keepdims=True))


def _tc2(u1, h1, asn, adn, A1, bg1, Wg2, a2src, a2dst, n):
    grid = n // BLK
    row = lambda i: (i, 0)
    full = lambda i: (0, 0)
    return pl.pallas_call(
        _tc2_body,
        grid=(grid,),
        in_specs=[
            pl.BlockSpec((8, BLK, ROWW), lambda i: (0, i, 0)),
            pl.BlockSpec((BLK, 128), row),
            pl.BlockSpec((BLK, 4), row), pl.BlockSpec((BLK, 4), row),
            pl.BlockSpec((1, 4), full), pl.BlockSpec((1, 128), full),
            pl.BlockSpec((128, 128), full),
            pl.BlockSpec((1, 128), full), pl.BlockSpec((1, 128), full),
        ],
        out_specs=[
            pl.BlockSpec((BLK, 128), row),
            pl.BlockSpec((BLK, ROWW), row), pl.BlockSpec((BLK, ROWW), row),
            pl.BlockSpec((BLK, ROWW), row), pl.BlockSpec((BLK, ROWW), row),
            pl.BlockSpec((BLK, 1), row), pl.BlockSpec((BLK, 1), row),
            pl.BlockSpec((1, 1), full),
        ],
        out_shape=[
            jax.ShapeDtypeStruct((n, 128), F32),
            jax.ShapeDtypeStruct((n, ROWW), F32), jax.ShapeDtypeStruct((n, ROWW), F32),
            jax.ShapeDtypeStruct((n, ROWW), F32), jax.ShapeDtypeStruct((n, ROWW), F32),
            jax.ShapeDtypeStruct((n, 1), F32), jax.ShapeDtypeStruct((n, 1), F32),
            jax.ShapeDtypeStruct((1, 1), F32),
        ],
    )(u1, h1, asn, adn, A1, bg1, Wg2, a2src, a2dst)


def _tc3_body(u_r, h2_r, as_r, ad_r, A_r, bg2_r, Wo1, bo1, Wo2, bo2, y_o):
    asn = as_r[...]
    adn = ad_r[...]
    z = asn + adn
    e = jnp.where(z > 0, z, 0.2 * z)
    zm = A_r[...] + adn
    m = jnp.where(zm > 0, zm, 0.2 * zm)
    ws = jnp.exp(e - m)
    u = u_r[...]
    h2 = h2_r[...]
    cols = []
    for c in range(4):
        uc = u[2 * c] + u[2 * c + 1]
        num = uc[:, 8:40] + ws * h2[:, 32 * c:32 * c + 32]
        den = uc[:, 0:1] + ws + 1e-16
        cols.append(num / den)
    x2 = jnp.concatenate(cols, axis=1) + bg2_r[...]
    x2 = _lr(jnp.dot(x2, Wo1[...], preferred_element_type=F32) + bo1[...], 0.01)
    y_o[...] = jnp.dot(x2, Wo2[...], preferred_element_type=F32) + bo2[...]


def _tc3(u2, h2, a2sn, a2dn, A2, bg2, Wo1, bo1, Wo2, bo2, n):
    grid = n // BLK
    row = lambda i: (i, 0)
    full = lambda i: (0, 0)
    return pl.pallas_call(
        _tc3_body,
        grid=(grid,),
        in_specs=[
            pl.BlockSpec((8, BLK, ROWW), lambda i: (0, i, 0)),
            pl.BlockSpec((BLK, 128), row),
            pl.BlockSpec((BLK, 1), row), pl.BlockSpec((BLK, 1), row),
            pl.BlockSpec((1, 1), full), pl.BlockSpec((1, 128), full),
            pl.BlockSpec((128, 128), full), pl.BlockSpec((1, 128), full),
            pl.BlockSpec((128, 2), full), pl.BlockSpec((1, 2), full),
        ],
        out_specs=[pl.BlockSpec((BLK, 2), row)],
        out_shape=[jax.ShapeDtypeStruct((n, 2), F32)],
    )(u2, h2, a2sn, a2dn, A2, bg2, Wo1, bo1, Wo2, bo2)[0]


def kernel(des, tweet, num_prop, cat_prop, edge_index,
           W_des, b_des, W_tweet, b_tweet, W_num, b_num, W_cat, b_cat,
           W_in, b_in, Wg1, a1_src, a1_dst, bg1, Wg2, a2_src, a2_dst, bg2,
           W_o1, b_o1, W_o2, b_o2):
    n = des.shape[0]
    E = edge_index.shape[1]
    rpt = -(-(n + 1) // NS)
    rpt = ((rpt + 7) // 8) * 8
    npad = rpt * NS
    ept = -(-E // (NW * BE)) * BE
    nblk = ept // BE
    e_pad = ept * NW

    r2 = lambda b: b.reshape(1, -1)
    h1, ht10, ht11, ht12, ht13, asn, adn, A1 = _tc1(
        des, tweet, num_prop, cat_prop,
        W_des, r2(b_des), W_tweet, r2(b_tweet), W_num, r2(b_num),
        W_cat, r2(b_cat), W_in, r2(b_in), Wg1, a1_src, a1_dst)

    src = edge_index[0]
    dst = edge_index[1]
    srcp = jnp.concatenate([src, jnp.zeros((e_pad - E,), jnp.int32)])
    dstp = jnp.concatenate([dst, jnp.full((e_pad - E,), n, jnp.int32)])
    zeros_t = jnp.zeros((rpt, ROWW), F32)

    def brd(col):
        t = jnp.broadcast_to(col.reshape(n, 1), (n, 16))
        return jnp.pad(t, ((0, npad - n), (0, 0)))

    AD1 = [brd(adn[:, h]) for h in range(4)]
    A1rep = jnp.broadcast_to(A1.reshape(4, 1), (4, 16))

    sc = _make_sc(n, npad, rpt, ept, nblk)
    u1 = sc(srcp, dstp, *AD1, ht10, ht11, ht12, ht13, A1rep, zeros_t)
    u1 = u1.reshape(8, npad, ROWW)

    h2, ht20, ht21, ht22, ht23, a2sn, a2dn, A2 = _tc2(
        u1, h1, asn, adn, A1, r2(bg1), Wg2, a2_src, a2_dst, n)

    AD2 = brd(a2dn)
    A2rep = jnp.broadcast_to(A2.reshape(1, 1), (4, 16))
    u2 = sc(srcp, dstp, AD2, AD2, AD2, AD2,
            ht20, ht21, ht22, ht23, A2rep, zeros_t)
    u2 = u2.reshape(8, npad, ROWW)

    return _tc3(u2, h2, a2sn, a2dn, A2, r2(bg2), W_o1, r2(b_o1),
                W_o2, r2(b_o2), n)


# speedup vs baseline: 1.2058x; 1.2058x over previous
import jax
import jax.numpy as jnp
from jax import lax
from jax.experimental import pallas as pl
from jax.experimental.pallas import tpu as pltpu
from jax.experimental.pallas import tpu_sc as plsc

F32 = jnp.float32
NC = 2
NS = 16
NW = NC * NS
BE = 64
ROWW = 40
BLK = 1000


def _lr(x, s):
    return jnp.where(x > 0, x, s * x)


def _tc1_body(des_r, tw_r, np_r, cp_r, Wd, bd, Wt, bt, Wn, bn, Wc, bc,
              Win, bin_, Wg1, a1s, a1d,
              h1_o, ht0_o, ht1_o, ht2_o, ht3_o, as_o, ad_o, amax_o):
    d = _lr(jnp.dot(des_r[...], Wd[...], preferred_element_type=F32) + bd[...], 0.01)
    t = _lr(jnp.dot(tw_r[...], Wt[...], preferred_element_type=F32) + bt[...], 0.01)
    npp = _lr(jnp.dot(np_r[...], Wn[...], preferred_element_type=F32) + bn[...], 0.01)
    c = _lr(jnp.dot(cp_r[...], Wc[...], preferred_element_type=F32) + bc[...], 0.01)
    x = jnp.concatenate([d, t, npp, c], axis=1)
    x = _lr(jnp.dot(x, Win[...], preferred_element_type=F32) + bin_[...], 0.01)
    h1 = jnp.dot(x, Wg1[...], preferred_element_type=F32)
    h1_o[...] = h1
    h1r = h1.reshape(-1, 4, 32)
    a_s = jnp.sum(h1r * a1s[...][None], axis=-1)
    a_d = jnp.sum(h1r * a1d[...][None], axis=-1)
    as_o[...] = a_s
    ad_o[...] = a_d
    one = jnp.ones((h1.shape[0], 1), F32)
    zer = jnp.zeros((h1.shape[0], 6), F32)
    for h, ref in enumerate((ht0_o, ht1_o, ht2_o, ht3_o)):
        ref[...] = jnp.concatenate(
            [one, a_s[:, h:h + 1], zer, h1[:, 32 * h:32 * h + 32]], axis=1)

    @pl.when(pl.program_id(0) == 0)
    def _():
        amax_o[...] = jnp.full((1, 4), -jnp.inf, F32)

    amax_o[...] = jnp.maximum(amax_o[...], jnp.max(a_s, axis=0)[None])


def _tc1(des, tweet, num_prop, cat_prop, Wd, bd, Wt, bt, Wn, bn, Wc, bc,
         Win, bin_, Wg1, a1s, a1d):
    n = des.shape[0]
    grid = n // BLK
    row = lambda i: (i, 0)
    full = lambda i: (0, 0)
    return pl.pallas_call(
        _tc1_body,
        grid=(grid,),
        in_specs=[
            pl.BlockSpec((BLK, 768), row), pl.BlockSpec((BLK, 768), row),
            pl.BlockSpec((BLK, 5), row), pl.BlockSpec((BLK, 3), row),
            pl.BlockSpec((768, 32), full), pl.BlockSpec((1, 32), full),
            pl.BlockSpec((768, 32), full), pl.BlockSpec((1, 32), full),
            pl.BlockSpec((5, 32), full), pl.BlockSpec((1, 32), full),
            pl.BlockSpec((3, 32), full), pl.BlockSpec((1, 32), full),
            pl.BlockSpec((128, 128), full), pl.BlockSpec((1, 128), full),
            pl.BlockSpec((128, 128), full),
            pl.BlockSpec((4, 32), full), pl.BlockSpec((4, 32), full),
        ],
        out_specs=[
            pl.BlockSpec((BLK, 128), row),
            pl.BlockSpec((BLK, ROWW), row), pl.BlockSpec((BLK, ROWW), row),
            pl.BlockSpec((BLK, ROWW), row), pl.BlockSpec((BLK, ROWW), row),
            pl.BlockSpec((BLK, 4), row), pl.BlockSpec((BLK, 4), row),
            pl.BlockSpec((1, 4), full),
        ],
        out_shape=[
            jax.ShapeDtypeStruct((n, 128), F32),
            jax.ShapeDtypeStruct((n, ROWW), F32), jax.ShapeDtypeStruct((n, ROWW), F32),
            jax.ShapeDtypeStruct((n, ROWW), F32), jax.ShapeDtypeStruct((n, ROWW), F32),
            jax.ShapeDtypeStruct((n, 4), F32), jax.ShapeDtypeStruct((n, 4), F32),
            jax.ShapeDtypeStruct((1, 4), F32),
        ],
    )(des, tweet, num_prop, cat_prop, Wd, bd, Wt, bt, Wn, bn, Wc, bc,
      Win, bin_, Wg1, a1s, a1d)


def _make_sc(n, npad, rpt, ept, nblk):
    def body(src_hbm, dst_hbm, ad0, ad1, ad2, ad3,
             ht0, ht1, ht2, ht3, a_hbm, z_hbm,
             out_hbm, a_v, src_v, dst_v, adb, rows_v, u_sh,
             sem1, sem3):
        cid = lax.axis_index("c")
        sid = lax.axis_index("s")
        wid = sid * NC + cid
        hts = (ht0, ht1, ht2, ht3)
        ads = (ad0, ad1, ad2, ad3)
        lt8 = jnp.arange(16, dtype=jnp.int32) < 8
        one16 = jnp.ones((16,), F32)
        for p in range(4):
            pltpu.sync_copy(a_hbm.at[p], a_v)
            pltpu.sync_copy(z_hbm, u_sh.at[pl.ds(sid * rpt, rpt)])
            plsc.subcore_barrier()
            av = a_v[...]

            def blk(b, carry):
                base = wid * ept + b * BE
                pltpu.sync_copy(src_hbm.at[pl.ds(base, BE)], src_v)
                pltpu.sync_copy(dst_hbm.at[pl.ds(base, BE)], dst_v)
                c1 = pltpu.async_copy(hts[p].at[src_v], rows_v, sem1)
                c3 = pltpu.async_copy(ads[p].at[dst_v], adb, sem3)
                c1.wait()
                c3.wait()
                for e in range(BE):
                    r0 = rows_v[e, pl.ds(0, 16)]
                    as16 = jnp.full((16,), r0[1], F32)
                    ad16 = adb[e, pl.ds(0, 16)]
                    z = as16 + ad16
                    e16 = jnp.where(z > 0, z, 0.2 * z)
                    zm = av + ad16
                    m16 = jnp.where(zm > 0, zm, 0.2 * zm)
                    wv = jnp.exp(e16 - m16)
                    wt = jnp.where(lt8, one16, wv)
                    rows_v[e, pl.ds(0, 16)] = r0 * wv
                    rows_v[e, pl.ds(16, 16)] = rows_v[e, pl.ds(16, 16)] * wv
                    rows_v[e, pl.ds(24, 16)] = rows_v[e, pl.ds(24, 16)] * wt
                pltpu.sync_copy(rows_v, u_sh.at[dst_v], add=True)
                return carry

            lax.fori_loop(0, nblk, blk, 0)
            plsc.subcore_barrier()
            pltpu.sync_copy(u_sh.at[pl.ds(sid * rpt, rpt)],
                            out_hbm.at[p, cid, pl.ds(sid * rpt, rpt)])
            plsc.subcore_barrier()

    return pl.kernel(
        body,
        out_type=jax.ShapeDtypeStruct((4, NC, npad, ROWW), F32),
        mesh=plsc.VectorSubcoreMesh(core_axis_name="c", subcore_axis_name="s",
                                    num_cores=NC, num_subcores=NS),
        compiler_params=pltpu.CompilerParams(needs_layout_passes=False,
                                             use_tc_tiling_on_sc=False),
        scratch_types=[
            pltpu.VMEM((16,), F32),
            pltpu.VMEM((BE,), jnp.int32), pltpu.VMEM((BE,), jnp.int32),
            pltpu.VMEM((BE, 16), F32),
            pltpu.VMEM((BE, ROWW), F32),
            pltpu.VMEM_SHARED((npad, ROWW), F32),
            pltpu.SemaphoreType.DMA, pltpu.SemaphoreType.DMA,
        ],
    )


def _tc2_body(u_r, h1_r, as_r, ad_r, A_r, bg1_r, Wg2, a2s_r, a2d_r,
              h2_o, ht0_o, ht1_o, ht2_o, ht3_o, a2so, a2do, amax_o):
    asn = as_r[...]
    adn = ad_r[...]
    z = asn + adn
    e = jnp.where(z > 0, z, 0.2 * z)
    zm = A_r[...] + adn
    m = jnp.where(zm > 0, zm, 0.2 * zm)
    ws = jnp.exp(e - m)
    u = u_r[...]
    h1 = h1_r[...]
    cols = []
    for h in range(4):
        uh = u[2 * h] + u[2 * h + 1]
        wsh = ws[:, h:h + 1]
        num = uh[:, 8:40] + wsh * h1[:, 32 * h:32 * h + 32]
        den = uh[:, 0:1] + wsh + 1e-16
        cols.append(num / den)
    x1 = jnp.concatenate(cols, axis=1) + bg1_r[...]
    h2 = jnp.dot(x1, Wg2[...], preferred_element_type=F32)
    h2_o[...] = h2
    a2s = jnp.sum(h2 * a2s_r[...], axis=1, keepdims=True)
    a2d = jnp.sum(h2 * a2d_r[...], axis=1, keepdims=True)
    a2so[...] = a2s
    a2do[...] = a2d
    one = jnp.ones((h2.shape[0], 1), F32)
    zer = jnp.zeros((h2.shape[0], 6), F32)
    for c, ref in enumerate((ht0_o, ht1_o, ht2_o, ht3_o)):
        ref[...] = jnp.concatenate(
            [one, a2s, zer, h2[:, 32 * c:32 * c + 32]], axis=1)

    @pl.when(pl.program_id(0) == 0)
    def _():
        amax_o[...] = jnp.full((1, 1), -jnp.inf, F32)

    amax_o[...] = jnp.maximum(amax_o[...], jnp.max(a2s, axis=0, keepdims=True))


def _tc2(u1, h1, asn, adn, A1, bg1, Wg2, a2src, a2dst, n):
    grid = n // BLK
    row = lambda i: (i, 0)
    full = lambda i: (0, 0)
    return pl.pallas_call(
        _tc2_body,
        grid=(grid,),
        in_specs=[
            pl.BlockSpec((8, BLK, ROWW), lambda i: (0, i, 0)),
            pl.BlockSpec((BLK, 128), row),
            pl.BlockSpec((BLK, 4), row), pl.BlockSpec((BLK, 4), row),
            pl.BlockSpec((1, 4), full), pl.BlockSpec((1, 128), full),
            pl.BlockSpec((128, 128), full),
            pl.BlockSpec((1, 128), full), pl.BlockSpec((1, 128), full),
        ],
        out_specs=[
            pl.BlockSpec((BLK, 128), row),
            pl.BlockSpec((BLK, ROWW), row), pl.BlockSpec((BLK, ROWW), row),
            pl.BlockSpec((BLK, ROWW), row), pl.BlockSpec((BLK, ROWW), row),
            pl.BlockSpec((BLK, 1), row), pl.BlockSpec((BLK, 1), row),
            pl.BlockSpec((1, 1), full),
        ],
        out_shape=[
            jax.ShapeDtypeStruct((n, 128), F32),
            jax.ShapeDtypeStruct((n, ROWW), F32), jax.ShapeDtypeStruct((n, ROWW), F32),
            jax.ShapeDtypeStruct((n, ROWW), F32), jax.ShapeDtypeStruct((n, ROWW), F32),
            jax.ShapeDtypeStruct((n, 1), F32), jax.ShapeDtypeStruct((n, 1), F32),
            jax.ShapeDtypeStruct((1, 1), F32),
        ],
    )(u1, h1, asn, adn, A1, bg1, Wg2, a2src, a2dst)


def _tc3_body(u_r, h2_r, as_r, ad_r, A_r, bg2_r, Wo1, bo1, Wo2, bo2, y_o):
    asn = as_r[...]
    adn = ad_r[...]
    z = asn + adn
    e = jnp.where(z > 0, z, 0.2 * z)
    zm = A_r[...] + adn
    m = jnp.where(zm > 0, zm, 0.2 * zm)
    ws = jnp.exp(e - m)
    u = u_r[...]
    h2 = h2_r[...]
    cols = []
    for c in range(4):
        uc = u[2 * c] + u[2 * c + 1]
        num = uc[:, 8:40] + ws * h2[:, 32 * c:32 * c + 32]
        den = uc[:, 0:1] + ws + 1e-16
        cols.append(num / den)
    x2 = jnp.concatenate(cols, axis=1) + bg2_r[...]
    x2 = _lr(jnp.dot(x2, Wo1[...], preferred_element_type=F32) + bo1[...], 0.01)
    y_o[...] = jnp.dot(x2, Wo2[...], preferred_element_type=F32) + bo2[...]


def _tc3(u2, h2, a2sn, a2dn, A2, bg2, Wo1, bo1, Wo2, bo2, n):
    grid = n // BLK
    row = lambda i: (i, 0)
    full = lambda i: (0, 0)
    return pl.pallas_call(
        _tc3_body,
        grid=(grid,),
        in_specs=[
            pl.BlockSpec((8, BLK, ROWW), lambda i: (0, i, 0)),
            pl.BlockSpec((BLK, 128), row),
            pl.BlockSpec((BLK, 1), row), pl.BlockSpec((BLK, 1), row),
            pl.BlockSpec((1, 1), full), pl.BlockSpec((1, 128), full),
            pl.BlockSpec((128, 128), full), pl.BlockSpec((1, 128), full),
            pl.BlockSpec((128, 2), full), pl.BlockSpec((1, 2), full),
        ],
        out_specs=[pl.BlockSpec((BLK, 2), row)],
        out_shape=[jax.ShapeDtypeStruct((n, 2), F32)],
    )(u2, h2, a2sn, a2dn, A2, bg2, Wo1, bo1, Wo2, bo2)[0]


def kernel(des, tweet, num_prop, cat_prop, edge_index,
           W_des, b_des, W_tweet, b_tweet, W_num, b_num, W_cat, b_cat,
           W_in, b_in, Wg1, a1_src, a1_dst, bg1, Wg2, a2_src, a2_dst, bg2,
           W_o1, b_o1, W_o2, b_o2):
    n = des.shape[0]
    E = edge_index.shape[1]
    rpt = -(-(n + 1) // NS)
    rpt = ((rpt + 7) // 8) * 8
    npad = rpt * NS
    ept = -(-E // (NW * BE)) * BE
    nblk = ept // BE
    e_pad = ept * NW

    r2 = lambda b: b.reshape(1, -1)
    h1, ht10, ht11, ht12, ht13, asn, adn, A1 = _tc1(
        des, tweet, num_prop, cat_prop,
        W_des, r2(b_des), W_tweet, r2(b_tweet), W_num, r2(b_num),
        W_cat, r2(b_cat), W_in, r2(b_in), Wg1, a1_src, a1_dst)

    src = edge_index[0]
    dst = edge_index[1]
    srcp = jnp.concatenate([src, jnp.zeros((e_pad - E,), jnp.int32)])
    dstp = jnp.concatenate([dst, jnp.full((e_pad - E,), n, jnp.int32)])
    zeros_t = jnp.zeros((rpt, ROWW), F32)

    def brd(col):
        t = jnp.broadcast_to(col.reshape(n, 1), (n, 16))
        return jnp.pad(t, ((0, npad - n), (0, 0)))

    AD1 = [brd(adn[:, h]) for h in range(4)]
    A1rep = jnp.broadcast_to(A1.reshape(4, 1), (4, 16))

    sc = _make_sc(n, npad, rpt, ept, nblk)
    u1 = sc(srcp, dstp, *AD1, ht10, ht11, ht12, ht13, A1rep, zeros_t)
    u1 = u1.reshape(8, npad, ROWW)

    h2, ht20, ht21, ht22, ht23, a2sn, a2dn, A2 = _tc2(
        u1, h1, asn, adn, A1, r2(bg1), Wg2, a2_src, a2_dst, n)

    AD2 = brd(a2dn)
    A2rep = jnp.broadcast_to(A2.reshape(1, 1), (4, 16))
    u2 = sc(srcp, dstp, AD2, AD2, AD2, AD2,
            ht20, ht21, ht22, ht23, A2rep, zeros_t)
    u2 = u2.reshape(8, npad, ROWW)

    return _tc3(u2, h2, a2sn, a2dn, A2, r2(bg2), W_o1, r2(b_o1),
                W_o2, r2(b_o2), n)


# speedup vs baseline: 1.2087x; 1.0024x over previous
import jax
import jax.numpy as jnp
from jax import lax
from jax.experimental import pallas as pl
from jax.experimental.pallas import tpu as pltpu
from jax.experimental.pallas import tpu_sc as plsc

F32 = jnp.float32
NC = 2
NS = 16
NW = NC * NS
BE = 64
ROWW = 40
BLK = 1000
BLK2 = 2000


def _lr(x, s):
    return jnp.where(x > 0, x, s * x)


def _tc1_body(des_r, tw_r, np_r, cp_r, Wd, bd, Wt, bt, Wn, bn, Wc, bc,
              Win, bin_, Wg1, a1s, a1d,
              h1_o, ht0_o, ht1_o, ht2_o, ht3_o, as_o, ad_o, amax_o):
    d = _lr(jnp.dot(des_r[...], Wd[...], preferred_element_type=F32) + bd[...], 0.01)
    t = _lr(jnp.dot(tw_r[...], Wt[...], preferred_element_type=F32) + bt[...], 0.01)
    npp = _lr(jnp.dot(np_r[...], Wn[...], preferred_element_type=F32) + bn[...], 0.01)
    c = _lr(jnp.dot(cp_r[...], Wc[...], preferred_element_type=F32) + bc[...], 0.01)
    x = jnp.concatenate([d, t, npp, c], axis=1)
    x = _lr(jnp.dot(x, Win[...], preferred_element_type=F32) + bin_[...], 0.01)
    h1 = jnp.dot(x, Wg1[...], preferred_element_type=F32)
    h1_o[...] = h1
    h1r = h1.reshape(-1, 4, 32)
    a_s = jnp.sum(h1r * a1s[...][None], axis=-1)
    a_d = jnp.sum(h1r * a1d[...][None], axis=-1)
    as_o[...] = a_s
    ad_o[...] = a_d
    one = jnp.ones((h1.shape[0], 1), F32)
    zer = jnp.zeros((h1.shape[0], 6), F32)
    for h, ref in enumerate((ht0_o, ht1_o, ht2_o, ht3_o)):
        ref[...] = jnp.concatenate(
            [one, a_s[:, h:h + 1], zer, h1[:, 32 * h:32 * h + 32]], axis=1)

    @pl.when(pl.program_id(0) == 0)
    def _():
        amax_o[...] = jnp.full((1, 4), -jnp.inf, F32)

    amax_o[...] = jnp.maximum(amax_o[...], jnp.max(a_s, axis=0)[None])


def _tc1(des, tweet, num_prop, cat_prop, Wd, bd, Wt, bt, Wn, bn, Wc, bc,
         Win, bin_, Wg1, a1s, a1d):
    n = des.shape[0]
    grid = n // BLK
    row = lambda i: (i, 0)
    full = lambda i: (0, 0)
    return pl.pallas_call(
        _tc1_body,
        grid=(grid,),
        in_specs=[
            pl.BlockSpec((BLK, 768), row), pl.BlockSpec((BLK, 768), row),
            pl.BlockSpec((BLK, 5), row), pl.BlockSpec((BLK, 3), row),
            pl.BlockSpec((768, 32), full), pl.BlockSpec((1, 32), full),
            pl.BlockSpec((768, 32), full), pl.BlockSpec((1, 32), full),
            pl.BlockSpec((5, 32), full), pl.BlockSpec((1, 32), full),
            pl.BlockSpec((3, 32), full), pl.BlockSpec((1, 32), full),
            pl.BlockSpec((128, 128), full), pl.BlockSpec((1, 128), full),
            pl.BlockSpec((128, 128), full),
            pl.BlockSpec((4, 32), full), pl.BlockSpec((4, 32), full),
        ],
        out_specs=[
            pl.BlockSpec((BLK, 128), row),
            pl.BlockSpec((BLK, ROWW), row), pl.BlockSpec((BLK, ROWW), row),
            pl.BlockSpec((BLK, ROWW), row), pl.BlockSpec((BLK, ROWW), row),
            pl.BlockSpec((BLK, 4), row), pl.BlockSpec((BLK, 4), row),
            pl.BlockSpec((1, 4), full),
        ],
        out_shape=[
            jax.ShapeDtypeStruct((n, 128), F32),
            jax.ShapeDtypeStruct((n, ROWW), F32), jax.ShapeDtypeStruct((n, ROWW), F32),
            jax.ShapeDtypeStruct((n, ROWW), F32), jax.ShapeDtypeStruct((n, ROWW), F32),
            jax.ShapeDtypeStruct((n, 4), F32), jax.ShapeDtypeStruct((n, 4), F32),
            jax.ShapeDtypeStruct((1, 4), F32),
        ],
    )(des, tweet, num_prop, cat_prop, Wd, bd, Wt, bt, Wn, bn, Wc, bc,
      Win, bin_, Wg1, a1s, a1d)


def _make_sc(n, npad, rpt, ept, nblk):
    def body(src_hbm, dst_hbm, ad0, ad1, ad2, ad3,
             ht0, ht1, ht2, ht3, a_hbm, z_hbm,
             out_hbm, a_v, src_v, dst_v, adb, rows_v, u_sh,
             sem1, sem3):
        cid = lax.axis_index("c")
        sid = lax.axis_index("s")
        wid = sid * NC + cid
        hts = (ht0, ht1, ht2, ht3)
        ads = (ad0, ad1, ad2, ad3)
        lt8 = jnp.arange(16, dtype=jnp.int32) < 8
        one16 = jnp.ones((16,), F32)
        for p in range(4):
            pltpu.sync_copy(a_hbm.at[p], a_v)
            pltpu.sync_copy(z_hbm, u_sh.at[pl.ds(sid * rpt, rpt)])
            plsc.subcore_barrier()
            av = a_v[...]

            def blk(b, carry):
                base = wid * ept + b * BE
                pltpu.sync_copy(src_hbm.at[pl.ds(base, BE)], src_v)
                pltpu.sync_copy(dst_hbm.at[pl.ds(base, BE)], dst_v)
                c1 = pltpu.async_copy(hts[p].at[src_v], rows_v, sem1)
                c3 = pltpu.async_copy(ads[p].at[dst_v], adb, sem3)
                c1.wait()
                c3.wait()
                for e in range(BE):
                    r0 = rows_v[e, pl.ds(0, 16)]
                    as16 = jnp.full((16,), r0[1], F32)
                    ad16 = adb[e, pl.ds(0, 16)]
                    z = as16 + ad16
                    e16 = jnp.where(z > 0, z, 0.2 * z)
                    zm = av + ad16
                    m16 = jnp.where(zm > 0, zm, 0.2 * zm)
                    wv = jnp.exp(e16 - m16)
                    wt = jnp.where(lt8, one16, wv)
                    rows_v[e, pl.ds(0, 16)] = r0 * wv
                    rows_v[e, pl.ds(16, 16)] = rows_v[e, pl.ds(16, 16)] * wv
                    rows_v[e, pl.ds(24, 16)] = rows_v[e, pl.ds(24, 16)] * wt
                pltpu.sync_copy(rows_v, u_sh.at[dst_v], add=True)
                return carry

            lax.fori_loop(0, nblk, blk, 0)
            plsc.subcore_barrier()
            pltpu.sync_copy(u_sh.at[pl.ds(sid * rpt, rpt)],
                            out_hbm.at[p, cid, pl.ds(sid * rpt, rpt)])
            plsc.subcore_barrier()

    return pl.kernel(
        body,
        out_type=jax.ShapeDtypeStruct((4, NC, npad, ROWW), F32),
        mesh=plsc.VectorSubcoreMesh(core_axis_name="c", subcore_axis_name="s",
                                    num_cores=NC, num_subcores=NS),
        compiler_params=pltpu.CompilerParams(needs_layout_passes=False,
                                             use_tc_tiling_on_sc=False),
        scratch_types=[
            pltpu.VMEM((16,), F32),
            pltpu.VMEM((BE,), jnp.int32), pltpu.VMEM((BE,), jnp.int32),
            pltpu.VMEM((BE, 16), F32),
            pltpu.VMEM((BE, ROWW), F32),
            pltpu.VMEM_SHARED((npad, ROWW), F32),
            pltpu.SemaphoreType.DMA, pltpu.SemaphoreType.DMA,
        ],
    )


def _tc2_body(u_r, h1_r, as_r, ad_r, A_r, bg1_r, Wg2, a2s_r, a2d_r,
              h2_o, ht0_o, ht1_o, ht2_o, ht3_o, a2so, a2do, amax_o):
    asn = as_r[...]
    adn = ad_r[...]
    z = asn + adn
    e = jnp.where(z > 0, z, 0.2 * z)
    zm = A_r[...] + adn
    m = jnp.where(zm > 0, zm, 0.2 * zm)
    ws = jnp.exp(e - m)
    u = u_r[...]
    h1 = h1_r[...]
    cols = []
    for h in range(4):
        uh = u[2 * h] + u[2 * h + 1]
        wsh = ws[:, h:h + 1]
        num = uh[:, 8:40] + wsh * h1[:, 32 * h:32 * h + 32]
        den = uh[:, 0:1] + wsh + 1e-16
        cols.append(num / den)
    x1 = jnp.concatenate(cols, axis=1) + bg1_r[...]
    h2 = jnp.dot(x1, Wg2[...], preferred_element_type=F32)
    h2_o[...] = h2
    a2s = jnp.sum(h2 * a2s_r[...], axis=1, keepdims=True)
    a2d = jnp.sum(h2 * a2d_r[...], axis=1, keepdims=True)
    a2so[...] = a2s
    a2do[...] = a2d
    one = jnp.ones((h2.shape[0], 1), F32)
    zer = jnp.zeros((h2.shape[0], 6), F32)
    for c, ref in enumerate((ht0_o, ht1_o, ht2_o, ht3_o)):
        ref[...] = jnp.concatenate(
            [one, a2s, zer, h2[:, 32 * c:32 * c + 32]], axis=1)

    @pl.when(pl.program_id(0) == 0)
    def _():
        amax_o[...] = jnp.full((1, 1), -jnp.inf, F32)

    amax_o[...] = jnp.maximum(amax_o[...], jnp.max(a2s, axis=0, keepdims=True))


def _tc2(u1, h1, asn, adn, A1, bg1, Wg2, a2src, a2dst, n):
    grid = n // BLK2
    row = lambda i: (i, 0)
    full = lambda i: (0, 0)
    return pl.pallas_call(
        _tc2_body,
        grid=(grid,),
        in_specs=[
            pl.BlockSpec((8, BLK2, ROWW), lambda i: (0, i, 0)),
            pl.BlockSpec((BLK2, 128), row),
            pl.BlockSpec((BLK2, 4), row), pl.BlockSpec((BLK2, 4), row),
            pl.BlockSpec((1, 4), full), pl.BlockSpec((1, 128), full),
            pl.BlockSpec((128, 128), full),
            pl.BlockSpec((1, 128), full), pl.BlockSpec((1, 128), full),
        ],
        out_specs=[
            pl.BlockSpec((BLK2, 128), row),
            pl.BlockSpec((BLK2, ROWW), row), pl.BlockSpec((BLK2, ROWW), row),
            pl.BlockSpec((BLK2, ROWW), row), pl.BlockSpec((BLK2, ROWW), row),
            pl.BlockSpec((BLK2, 1), row), pl.BlockSpec((BLK2, 1), row),
            pl.BlockSpec((1, 1), full),
        ],
        out_shape=[
            jax.ShapeDtypeStruct((n, 128), F32),
            jax.ShapeDtypeStruct((n, ROWW), F32), jax.ShapeDtypeStruct((n, ROWW), F32),
            jax.ShapeDtypeStruct((n, ROWW), F32), jax.ShapeDtypeStruct((n, ROWW), F32),
            jax.ShapeDtypeStruct((n, 1), F32), jax.ShapeDtypeStruct((n, 1), F32),
            jax.ShapeDtypeStruct((1, 1), F32),
        ],
    )(u1, h1, asn, adn, A1, bg1, Wg2, a2src, a2dst)


def _tc3_body(u_r, h2_r, as_r, ad_r, A_r, bg2_r, Wo1, bo1, Wo2, bo2, y_o):
    asn = as_r[...]
    adn = ad_r[...]
    z = asn + adn
    e = jnp.where(z > 0, z, 0.2 * z)
    zm = A_r[...] + adn
    m = jnp.where(zm > 0, zm, 0.2 * zm)
    ws = jnp.exp(e - m)
    u = u_r[...]
    h2 = h2_r[...]
    cols = []
    for c in range(4):
        uc = u[2 * c] + u[2 * c + 1]
        num = uc[:, 8:40] + ws * h2[:, 32 * c:32 * c + 32]
        den = uc[:, 0:1] + ws + 1e-16
        cols.append(num / den)
    x2 = jnp.concatenate(cols, axis=1) + bg2_r[...]
    x2 = _lr(jnp.dot(x2, Wo1[...], preferred_element_type=F32) + bo1[...], 0.01)
    y_o[...] = jnp.dot(x2, Wo2[...], preferred_element_type=F32) + bo2[...]


def _tc3(u2, h2, a2sn, a2dn, A2, bg2, Wo1, bo1, Wo2, bo2, n):
    grid = n // BLK2
    row = lambda i: (i, 0)
    full = lambda i: (0, 0)
    return pl.pallas_call(
        _tc3_body,
        grid=(grid,),
        in_specs=[
            pl.BlockSpec((8, BLK2, ROWW), lambda i: (0, i, 0)),
            pl.BlockSpec((BLK2, 128), row),
            pl.BlockSpec((BLK2, 1), row), pl.BlockSpec((BLK2, 1), row),
            pl.BlockSpec((1, 1), full), pl.BlockSpec((1, 128), full),
            pl.BlockSpec((128, 128), full), pl.BlockSpec((1, 128), full),
            pl.BlockSpec((128, 2), full), pl.BlockSpec((1, 2), full),
        ],
        out_specs=[pl.BlockSpec((BLK2, 2), row)],
        out_shape=[jax.ShapeDtypeStruct((n, 2), F32)],
    )(u2, h2, a2sn, a2dn, A2, bg2, Wo1, bo1, Wo2, bo2)[0]


def kernel(des, tweet, num_prop, cat_prop, edge_index,
           W_des, b_des, W_tweet, b_tweet, W_num, b_num, W_cat, b_cat,
           W_in, b_in, Wg1, a1_src, a1_dst, bg1, Wg2, a2_src, a2_dst, bg2,
           W_o1, b_o1, W_o2, b_o2):
    n = des.shape[0]
    E = edge_index.shape[1]
    rpt = -(-(n + 1) // NS)
    rpt = ((rpt + 7) // 8) * 8
    npad = rpt * NS
    ept = -(-E // (NW * BE)) * BE
    nblk = ept // BE
    e_pad = ept * NW

    r2 = lambda b: b.reshape(1, -1)
    h1, ht10, ht11, ht12, ht13, asn, adn, A1 = _tc1(
        des, tweet, num_prop, cat_prop,
        W_des, r2(b_des), W_tweet, r2(b_tweet), W_num, r2(b_num),
        W_cat, r2(b_cat), W_in, r2(b_in), Wg1, a1_src, a1_dst)

    src = edge_index[0]
    dst = edge_index[1]
    srcp = jnp.concatenate([src, jnp.zeros((e_pad - E,), jnp.int32)])
    dstp = jnp.concatenate([dst, jnp.full((e_pad - E,), n, jnp.int32)])
    zeros_t = jnp.zeros((rpt, ROWW), F32)

    def brd(col):
        t = jnp.broadcast_to(col.reshape(n, 1), (n, 16))
        return jnp.pad(t, ((0, npad - n), (0, 0)))

    AD1 = [brd(adn[:, h]) for h in range(4)]
    A1rep = jnp.broadcast_to(A1.reshape(4, 1), (4, 16))

    sc = _make_sc(n, npad, rpt, ept, nblk)
    u1 = sc(srcp, dstp, *AD1, ht10, ht11, ht12, ht13, A1rep, zeros_t)
    u1 = u1.reshape(8, npad, ROWW)

    h2, ht20, ht21, ht22, ht23, a2sn, a2dn, A2 = _tc2(
        u1, h1, asn, adn, A1, r2(bg1), Wg2, a2_src, a2_dst, n)

    AD2 = brd(a2dn)
    A2rep = jnp.broadcast_to(A2.reshape(1, 1), (4, 16))
    u2 = sc(srcp, dstp, AD2, AD2, AD2, AD2,
            ht20, ht21, ht22, ht23, A2rep, zeros_t)
    u2 = u2.reshape(8, npad, ROWW)

    return _tc3(u2, h2, a2sn, a2dn, A2, r2(bg2), W_o1, r2(b_o1),
                W_o2, r2(b_o2), n)
